# baseline (speedup 1.0000x reference)
import functools
import math

import jax
import jax.numpy as jnp
from jax import lax
from jax.experimental import pallas as pl
from jax.experimental.pallas import tpu as pltpu
from jax.experimental.pallas import tpu_sc as plsc

_D = 64
_SCALE = 8.0
_LANES = 16
_NDMA = 8


@functools.cache
def _make_sc_gather(B, V, D, chunk):
    NC, NS = 2, 16
    NW = NC * NS
    b_per_w = B // NW
    assert b_per_w * NW == B and b_per_w % chunk == 0
    n_chunks = b_per_w // chunk
    mesh = plsc.VectorSubcoreMesh(core_axis_name="c", subcore_axis_name="s")

    @functools.partial(
        pl.kernel,
        out_type=jax.ShapeDtypeStruct((B, D), jnp.float32),
        mesh=mesh,
        scratch_types=[
            pltpu.VMEM((chunk,), jnp.int32),
            pltpu.VMEM((chunk,), jnp.int32),
            pltpu.VMEM((chunk, D), jnp.float32),
            pltpu.VMEM((chunk, D), jnp.float32),
            pltpu.SemaphoreType.DMA,
            pltpu.SemaphoreType.DMA,
            pltpu.SemaphoreType.DMA,
            pltpu.SemaphoreType.DMA,
        ],
        compiler_params=pltpu.CompilerParams(use_tc_tiling_on_sc=False),
    )
    def sc_gather(x_hbm, table_hbm, out_hbm, idx0, idx1, rows0, rows1,
                  sg0, sg1, ss0, ss1):
        wid = lax.axis_index("s") * NC + lax.axis_index("c")
        base = wid * b_per_w
        slots = ((idx0, rows0, sg0, ss0), (idx1, rows1, sg1, ss1))

        def start_gather(g, slot):
            idx, rows, sg, _ = slot
            pltpu.sync_copy(x_hbm.at[pl.ds(base + g * chunk, chunk)], idx)
            pltpu.async_copy(table_hbm.at[idx], rows, sg)

        def wait_gather(slot):
            idx, rows, sg, _ = slot
            pltpu.make_async_copy(table_hbm.at[idx], rows, sg).wait()

        def scale(slot):
            rows = slot[1]

            def row_body(r, _):
                for j in range(D // _LANES):
                    sl = pl.ds(j * _LANES, _LANES)
                    rows[r, sl] = rows[r, sl] * _SCALE
                return ()

            lax.fori_loop(0, chunk, row_body, (), unroll=8)

        def start_scatter(g, slot):
            _, rows, _, ss = slot
            pltpu.async_copy(rows, out_hbm.at[pl.ds(base + g * chunk, chunk)], ss)

        def wait_scatter(g, slot):
            _, rows, _, ss = slot
            pltpu.make_async_copy(
                rows, out_hbm.at[pl.ds(base + g * chunk, chunk)], ss).wait()

        start_gather(0, slots[0])

        def pair(p, _):
            for b in range(2):
                g = p * 2 + b
                nslot = slots[1 - b]

                @pl.when(g + 1 < n_chunks)
                def _():
                    @pl.when(g >= 1)
                    def _():
                        wait_scatter(g - 1, nslot)

                    start_gather(g + 1, nslot)

                wait_gather(slots[b])
                scale(slots[b])
                start_scatter(g, slots[b])
            return ()

        lax.fori_loop(0, n_chunks // 2, pair, ())
        wait_scatter(n_chunks - 2, slots[0])
        wait_scatter(n_chunks - 1, slots[1])

    return sc_gather


@functools.cache
def _make_tc_copy(src_shape, dst_shape, n_slices):
    tot = 1
    for s in src_shape:
        tot *= s
    rows = tot // _D
    assert rows % n_slices == 0
    R = rows // n_slices

    def v2(ref, shape):
        return ref if len(shape) == 2 and shape[1] == _D \
            else ref.reshape(rows, _D)

    def body(i_ref, o_ref, buf0, buf1, sems):
        i2 = v2(i_ref, src_shape)
        o2 = v2(o_ref, dst_shape)
        bufs = (buf0, buf1)

        def load(k):
            return pltpu.make_async_copy(
                i2.at[pl.ds(k * R, R)], bufs[k % 2], sems.at[k % 2])

        def store(k):
            return pltpu.make_async_copy(
                bufs[k % 2], o2.at[pl.ds(k * R, R)], sems.at[2 + k % 2])

        load(0).start()
        for k in range(n_slices):
            load(k).wait()
            if k + 1 < n_slices:
                if k >= 1:
                    store(k - 1).wait()
                load(k + 1).start()
            store(k).start()
        store(n_slices - 2).wait()
        store(n_slices - 1).wait()

    return pl.pallas_call(
        body,
        in_specs=[pl.BlockSpec(memory_space=pl.ANY)],
        out_specs=pl.BlockSpec(memory_space=pl.ANY),
        out_shape=jax.ShapeDtypeStruct(dst_shape, jnp.float32),
        scratch_shapes=[
            pltpu.VMEM((R, _D), jnp.float32),
            pltpu.VMEM((R, _D), jnp.float32),
            pltpu.SemaphoreType.DMA((4,)),
        ],
    )


def kernel(x, table):
    S0, S1 = x.shape
    V, D = table.shape
    B = S0 * S1
    flat = x.reshape(B).astype(jnp.int32)
    table_lin = _make_tc_copy((V, D), (V, D), 32)(table)
    y = _make_sc_gather(B, V, D, 800)(flat, table_lin)
    return _make_tc_copy((B, D), (S0, S1, D), 64)(y)


# speedup vs baseline: 1.6346x; 1.6346x over previous
import functools
import math

import jax
import jax.numpy as jnp
from jax import lax
from jax.experimental import pallas as pl
from jax.experimental.pallas import tpu as pltpu
from jax.experimental.pallas import tpu_sc as plsc

_D = 64
_SCALE = 8.0
_LANES = 16


@functools.cache
def _make_sc_gather(S0, S1, V, D, chunk):
    B = S0 * S1
    NC, NS = 2, 16
    NW = NC * NS
    b_per_w = B // NW
    assert b_per_w * NW == B and b_per_w % chunk == 0
    assert chunk % S1 == 0
    rows_per_chunk = chunk // S1
    n_chunks = b_per_w // chunk
    mesh = plsc.VectorSubcoreMesh(core_axis_name="c", subcore_axis_name="s")

    @functools.partial(
        pl.kernel,
        out_type=jax.ShapeDtypeStruct((S0, S1, D), jnp.float32),
        mesh=mesh,
        scratch_types=[
            pltpu.VMEM((chunk,), jnp.int32),
            pltpu.VMEM((chunk,), jnp.int32),
            pltpu.VMEM((chunk, D), jnp.float32),
            pltpu.VMEM((chunk, D), jnp.float32),
            pltpu.SemaphoreType.DMA,
            pltpu.SemaphoreType.DMA,
            pltpu.SemaphoreType.DMA,
            pltpu.SemaphoreType.DMA,
        ],
        compiler_params=pltpu.CompilerParams(use_tc_tiling_on_sc=False),
    )
    def sc_gather(x_hbm, table_hbm, out_hbm, idx0, idx1, rows0, rows1,
                  sg0, sg1, ss0, ss1):
        wid = lax.axis_index("s") * NC + lax.axis_index("c")
        base = wid * b_per_w
        slots = ((idx0, rows0, sg0, ss0), (idx1, rows1, sg1, ss1))

        def start_gather(g, slot):
            idx, rows, sg, _ = slot
            pltpu.sync_copy(x_hbm.at[pl.ds(base + g * chunk, chunk)], idx)
            pltpu.async_copy(table_hbm.at[idx], rows, sg)

        def wait_gather(slot):
            idx, rows, sg, _ = slot
            pltpu.make_async_copy(table_hbm.at[idx], rows, sg).wait()

        def scale(slot):
            rows = slot[1]

            def row_body(r, _):
                for j in range(D // _LANES):
                    sl = pl.ds(j * _LANES, _LANES)
                    rows[r, sl] = rows[r, sl] * _SCALE
                return ()

            lax.fori_loop(0, chunk, row_body, (), unroll=8)

        def start_scatter(g, slot):
            _, rows, _, ss = slot
            xr0 = (base + g * chunk) // S1
            for k in range(rows_per_chunk):
                pltpu.async_copy(
                    rows.at[pl.ds(k * S1, S1)], out_hbm.at[xr0 + k], ss)

        def wait_scatter(g, slot):
            _, rows, _, ss = slot
            xr0 = (base + g * chunk) // S1
            for k in range(rows_per_chunk):
                pltpu.make_async_copy(
                    rows.at[pl.ds(k * S1, S1)], out_hbm.at[xr0 + k], ss).wait()

        start_gather(0, slots[0])

        def pair(p, _):
            for b in range(2):
                g = p * 2 + b
                nslot = slots[1 - b]

                @pl.when(g + 1 < n_chunks)
                def _():
                    @pl.when(g >= 1)
                    def _():
                        wait_scatter(g - 1, nslot)

                    start_gather(g + 1, nslot)

                wait_gather(slots[b])
                scale(slots[b])
                start_scatter(g, slots[b])
            return ()

        lax.fori_loop(0, n_chunks // 2, pair, ())
        wait_scatter(n_chunks - 2, slots[0])
        wait_scatter(n_chunks - 1, slots[1])

    return sc_gather


def kernel(x, table):
    S0, S1 = x.shape
    V, D = table.shape
    flat = x.reshape(S0 * S1).astype(jnp.int32)
    return _make_sc_gather(S0, S1, V, D, 800)(flat, table)
